# baseline (speedup 1.0000x reference)
import functools

import jax
import jax.numpy as jnp
from jax import lax
from jax.experimental import pallas as pl
from jax.experimental.pallas import tpu as pltpu
from jax.experimental.pallas import tpu_sc as plsc

D = 128
NC, NS = 2, 16
NW = NC * NS
BB = 4
NBUF = 4


def _make_gather(b: int, s: int):
    ch = BB * s
    assert b % (NW * BB * NBUF) == 0
    b_per_w = b // NW
    chunks = b_per_w // BB
    n_pass = chunks // NBUF
    mesh = plsc.VectorSubcoreMesh(core_axis_name="c", subcore_axis_name="s")

    @functools.partial(
        pl.kernel,
        mesh=mesh,
        out_type=jax.ShapeDtypeStruct((b, s, D), jnp.float32),
        scratch_types=[
            pltpu.VMEM((chunks, ch), jnp.int32),
            pltpu.VMEM((NBUF, ch, D), jnp.float32),
        ]
        + [pltpu.SemaphoreType.DMA] * (2 * NBUF),
        compiler_params=pltpu.CompilerParams(use_tc_tiling_on_sc=True),
    )
    def gather(table_hbm, idx_hbm, out_hbm, idx_v, rows_v, *sems):
        gsem, wsem = sems[:NBUF], sems[NBUF:]
        wid = lax.axis_index("s") * NC + lax.axis_index("c")
        pltpu.sync_copy(idx_hbm.at[wid], idx_v)
        base_b = wid * b_per_w

        def g_copy(chunk, buf):
            return pltpu.make_async_copy(
                table_hbm.at[idx_v.at[chunk]], rows_v.at[buf], gsem[buf]
            )

        def w_copies(chunk, buf):
            b0 = base_b + chunk * BB
            return [
                pltpu.make_async_copy(
                    rows_v.at[buf, pl.ds(k * s, s)], out_hbm.at[b0 + k], wsem[buf]
                )
                for k in range(BB)
            ]

        for buf in range(NBUF):
            g_copy(buf, buf).start()

        def pass_body(i, carry):
            j = i * NBUF
            for buf in range(NBUF):
                g_copy(j + buf, buf).wait()
                for c in w_copies(j + buf, buf):
                    c.start()

            @pl.when(i < n_pass - 1)
            def _():
                for buf in range(NBUF):
                    for c in w_copies(j + buf, buf):
                        c.wait()
                    g_copy(j + NBUF + buf, buf).start()

            return carry

        lax.fori_loop(0, n_pass, pass_body, 0)
        for buf in range(NBUF):
            for c in w_copies(chunks - NBUF + buf, buf):
                c.wait()

    return gather


def kernel(input, indices):
    b, s = indices.shape
    idx = indices.reshape(NW, b // (NW * BB), BB * s).astype(jnp.int32)
    return _make_gather(b, s)(input, idx)
